# baseline (speedup 1.0000x reference)
import jax, jax.numpy as jnp
from jax.experimental import pallas as pl

NG = 64


def _gcn_conv(x, ei, W, b, em, nm):
    n = x.shape[0]
    sl = jnp.arange(n)
    src = jnp.concatenate([ei[0], sl])
    dst = jnp.concatenate([ei[1], sl])
    w = jnp.concatenate([em, nm]).astype(x.dtype)
    deg = jnp.zeros((n,), x.dtype).at[dst].add(w)
    dinv = jnp.where(deg > 0, deg ** -0.5, 0.0)
    norm = (dinv[src] * dinv[dst] * w)[:, None]
    h = x @ W
    out = jnp.zeros((n, W.shape[1]), x.dtype).at[dst].add(norm * h[src])
    return out + b


def _topk_pool(h, s, bt, nm, start, num_graphs):
    n = h.shape[0]
    idx = jnp.arange(n)
    key = jnp.where(nm, -s, jnp.inf)
    order = jnp.lexsort((idx, key, bt))
    og = bt[order]
    rank = idx - start[og]
    c = jax.ops.segment_sum(nm.astype(jnp.int32), bt, num_segments=num_graphs)
    k = (4 * c + 4) // 5
    kept = (rank < k[og]) & nm[order]
    hp = jnp.where(kept[:, None], h[order] * jnp.tanh(s[order])[:, None], jnp.zeros((), h.dtype))
    inv = jnp.zeros((n,), idx.dtype).at[order].set(idx)
    kept_prev = jnp.zeros((n,), bool).at[order].set(kept)
    return hp, og, kept, inv, kept_prev


def _readout(x, batch, G, nm):
    neg = jnp.where(nm[:, None], x, -jnp.inf)
    mx = jax.ops.segment_max(neg, batch, num_segments=G)
    xm = jnp.where(nm[:, None], x, jnp.zeros((), x.dtype))
    sm = jax.ops.segment_sum(xm, batch, num_segments=G)
    cnt = jax.ops.segment_sum(nm.astype(x.dtype), batch, num_segments=G)
    return jnp.concatenate([mx, sm / cnt[:, None]], axis=1)


def _id_kernel(x_ref, o_ref):
    o_ref[...] = x_ref[...]


def kernel(x, edge_index, batch, W1, b1, Wp1, bp1, W2, b2, Wp2, bp2, W3, b3, Wp3, bp3, Wl1, bl1, Wl2, bl2, Wl3, bl3):
    G = NG
    n = x.shape[0]
    ei = edge_index
    bt = batch
    nm = jnp.ones((n,), bool)
    em = jnp.ones((ei.shape[1],), bool)
    cnt_all = jax.ops.segment_sum(jnp.ones((n,), jnp.int32), bt, num_segments=G)
    start = jnp.cumsum(cnt_all) - cnt_all
    h = jax.nn.relu(_gcn_conv(x, ei, W1, b1, em, nm))
    s = _gcn_conv(h, ei, Wp1, bp1, em, nm)[:, 0]
    h, bt, nm_new, inv, kept_prev = _topk_pool(h, s, bt, nm, start, G)
    em = em & kept_prev[ei[0]] & kept_prev[ei[1]]
    ei = jnp.stack([inv[ei[0]], inv[ei[1]]])
    nm = nm_new
    x1 = _readout(h, bt, G, nm)
    h = jax.nn.relu(_gcn_conv(h, ei, W2, b2, em, nm))
    s = _gcn_conv(h, ei, Wp2, bp2, em, nm)[:, 0]
    h, bt, nm_new, inv, kept_prev = _topk_pool(h, s, bt, nm, start, G)
    em = em & kept_prev[ei[0]] & kept_prev[ei[1]]
    ei = jnp.stack([inv[ei[0]], inv[ei[1]]])
    nm = nm_new
    x2 = _readout(h, bt, G, nm)
    h = jax.nn.relu(_gcn_conv(h, ei, W3, b3, em, nm))
    s = _gcn_conv(h, ei, Wp3, bp3, em, nm)[:, 0]
    h, bt, nm, inv, kept_prev = _topk_pool(h, s, bt, nm, start, G)
    x3 = _readout(h, bt, G, nm)
    z = x1 + x2 + x3
    z = pl.pallas_call(
        _id_kernel,
        out_shape=jax.ShapeDtypeStruct(z.shape, z.dtype),
    )(z)
    z = jax.nn.relu(z @ Wl1 + bl1)
    z = jax.nn.relu(z @ Wl2 + bl2)
    return jax.nn.log_softmax(z @ Wl3 + bl3, axis=-1)


# speedup vs baseline: 21.3813x; 21.3813x over previous
import functools

import jax
import jax.numpy as jnp
from jax import lax
from jax.experimental import pallas as pl
from jax.experimental.pallas import tpu as pltpu
from jax.experimental.pallas import tpu_sc as plsc

N = 10000
D = 128
NG = 64
NP = 10112
NB = NP // 128
E = 320000
NW = 32
CH = 81
CW = 128
EP = NW * CH * CW
EW = CH * CW
ACC_H = 10240
TRASH = NP
F32 = jnp.float32
I32 = jnp.int32

_MESH = dict(core_axis_name="c", subcore_axis_name="s", num_cores=2,
             num_subcores=16)


def _k1_body(src_h, pm_h, d16_h, kept_h, z16_h, wo_h, degA_h, degB_h,
             kept_v, src_v, pm_v, wo_v, d16_v, rows16, acc):
    cid = lax.axis_index("c")
    sid = lax.axis_index("s")
    wid = cid * 16 + sid
    pltpu.sync_copy(z16_h.at[pl.ds(sid * 640, 640)], acc.at[pl.ds(sid * 640, 640)])
    pltpu.sync_copy(z16_h.at[pl.ds(0, 16)], rows16)
    pltpu.sync_copy(kept_h, kept_v)
    pltpu.sync_copy(src_h.at[wid], src_v)
    pltpu.sync_copy(pm_h.at[wid], pm_v)
    pltpu.sync_copy(d16_h.at[wid], d16_v)
    plsc.subcore_barrier()
    lanes = lax.iota(I32, 16)

    def group(g, _):
        o = g * 16
        sv = src_v[pl.ds(o, 16)]
        dv = d16_v[g]
        pm = pm_v[pl.ds(o, 16)]
        ks = plsc.load_gather(kept_v, [sv])
        kd = plsc.load_gather(kept_v, [dv])
        wn = pm * ks * kd
        wo_v[pl.ds(o, 16)] = wn
        plsc.store_scatter(rows16, [lanes, jnp.zeros((16,), I32)], wn)
        pltpu.sync_copy(rows16, acc.at[d16_v.at[g]], add=True)
        return 0

    lax.fori_loop(0, EW // 16, group, 0)
    pltpu.sync_copy(wo_v, wo_h.at[wid])
    plsc.subcore_barrier()

    @pl.when(jnp.logical_and(sid == 0, cid == 0))
    def _():
        pltpu.sync_copy(acc, degA_h)

    @pl.when(jnp.logical_and(sid == 0, cid == 1))
    def _():
        pltpu.sync_copy(acc, degB_h)


def _sc_k1(src3, pm3, d16, kept, z16):
    fn = pl.kernel(
        _k1_body,
        out_type=(
            jax.ShapeDtypeStruct((NW, EW), F32),
            jax.ShapeDtypeStruct((ACC_H, 16), F32),
            jax.ShapeDtypeStruct((ACC_H, 16), F32),
        ),
        mesh=plsc.VectorSubcoreMesh(**_MESH),
        compiler_params=pltpu.CompilerParams(needs_layout_passes=False, use_tc_tiling_on_sc=False),
        scratch_types=(
            pltpu.VMEM((NP,), F32),
            pltpu.VMEM((EW,), I32),
            pltpu.VMEM((EW,), F32),
            pltpu.VMEM((EW,), F32),
            pltpu.VMEM((EW // 16, 16), I32),
            pltpu.VMEM((16, 16), F32),
            pltpu.VMEM_SHARED((ACC_H, 16), F32),
        ),
    )
    return fn(src3, pm3, d16, kept, z16)


def _k2_body(hws_h, src_h, dst_h, z128_h, aggA_h, aggB_h,
             src_v, dst_v, rows_v, acc, sem):
    cid = lax.axis_index("c")
    sid = lax.axis_index("s")
    wid = cid * 16 + sid
    pltpu.sync_copy(z128_h.at[pl.ds(sid * 640, 640)], acc.at[pl.ds(sid * 640, 640)])
    pltpu.sync_copy(src_h.at[wid], src_v)
    pltpu.sync_copy(dst_h.at[wid], dst_v)
    plsc.subcore_barrier()

    def chunk(c, _):
        pltpu.async_copy(hws_h.at[src_v.at[c]], rows_v, sem).wait()
        pltpu.sync_copy(rows_v, acc.at[dst_v.at[c]], add=True)
        return 0

    lax.fori_loop(0, CH, chunk, 0)
    plsc.subcore_barrier()

    @pl.when(jnp.logical_and(sid == 0, cid == 0))
    def _():
        pltpu.sync_copy(acc, aggA_h)

    @pl.when(jnp.logical_and(sid == 0, cid == 1))
    def _():
        pltpu.sync_copy(acc, aggB_h)


def _sc_k2(hws, src3c, dst3c, z128):
    fn = pl.kernel(
        _k2_body,
        out_type=(
            jax.ShapeDtypeStruct((ACC_H, D), F32),
            jax.ShapeDtypeStruct((ACC_H, D), F32),
        ),
        mesh=plsc.VectorSubcoreMesh(**_MESH),
        compiler_params=pltpu.CompilerParams(needs_layout_passes=False, use_tc_tiling_on_sc=False),
        scratch_types=(
            pltpu.VMEM((CH, CW), I32),
            pltpu.VMEM((CH, CW), I32),
            pltpu.VMEM((CW, D), F32),
            pltpu.VMEM_SHARED((ACC_H, D), F32),
            pltpu.SemaphoreType.DMA,
        ),
    )
    return fn(hws, src3c, dst3c, z128)


def _k3_body(tab_h, src_h, w_h, d16_h, z16_h, sA_h, sB_h,
             tab_v, src_v, w_v, d16_v, rows16, acc):
    cid = lax.axis_index("c")
    sid = lax.axis_index("s")
    wid = cid * 16 + sid
    pltpu.sync_copy(z16_h.at[pl.ds(sid * 640, 640)], acc.at[pl.ds(sid * 640, 640)])
    pltpu.sync_copy(z16_h.at[pl.ds(0, 16)], rows16)
    pltpu.sync_copy(tab_h, tab_v)
    pltpu.sync_copy(src_h.at[wid], src_v)
    pltpu.sync_copy(w_h.at[wid], w_v)
    pltpu.sync_copy(d16_h.at[wid], d16_v)
    plsc.subcore_barrier()
    lanes = lax.iota(I32, 16)

    def group(g, _):
        o = g * 16
        sv = src_v[pl.ds(o, 16)]
        wv = w_v[pl.ds(o, 16)]
        val = wv * plsc.load_gather(tab_v, [sv])
        plsc.store_scatter(rows16, [lanes, jnp.zeros((16,), I32)], val)
        pltpu.sync_copy(rows16, acc.at[d16_v.at[g]], add=True)
        return 0

    lax.fori_loop(0, EW // 16, group, 0)
    plsc.subcore_barrier()

    @pl.when(jnp.logical_and(sid == 0, cid == 0))
    def _():
        pltpu.sync_copy(acc, sA_h)

    @pl.when(jnp.logical_and(sid == 0, cid == 1))
    def _():
        pltpu.sync_copy(acc, sB_h)


def _sc_k3(tab, src3, w3, d16, z16):
    fn = pl.kernel(
        _k3_body,
        out_type=(
            jax.ShapeDtypeStruct((ACC_H, 16), F32),
            jax.ShapeDtypeStruct((ACC_H, 16), F32),
        ),
        mesh=plsc.VectorSubcoreMesh(**_MESH),
        compiler_params=pltpu.CompilerParams(needs_layout_passes=False, use_tc_tiling_on_sc=False),
        scratch_types=(
            pltpu.VMEM((NP,), F32),
            pltpu.VMEM((EW,), I32),
            pltpu.VMEM((EW,), F32),
            pltpu.VMEM((EW // 16, 16), I32),
            pltpu.VMEM((16, 16), F32),
            pltpu.VMEM_SHARED((ACC_H, 16), F32),
        ),
    )
    return fn(tab, src3, w3, d16, z16)


def _counts_body(btf_ref, kept_ref, cnt_ref):
    i = pl.program_id(0)

    @pl.when(i == 0)
    def _():
        cnt_ref[...] = jnp.zeros_like(cnt_ref)

    btr = btf_ref[...]
    kr = kept_ref[...]
    for g in range(NG):
        val = jnp.sum(jnp.where(btr == float(g), kr, 0.0))
        cnt_ref[g:g + 1, :] = cnt_ref[g:g + 1, :] + val


def _tc_counts(btf, kept):
    return pl.pallas_call(
        _counts_body,
        grid=(NB,),
        in_specs=[
            pl.BlockSpec((128, 1), lambda i: (i, 0)),
            pl.BlockSpec((128, 1), lambda i: (i, 0)),
        ],
        out_specs=pl.BlockSpec((NG, 128), lambda i: (0, 0)),
        out_shape=jax.ShapeDtypeStruct((NG, 128), F32),
    )(btf, kept)


def _matscale_body(x_ref, w_ref, dinv_ref, o_ref):
    o_ref[...] = dinv_ref[...] * jnp.dot(
        x_ref[...], w_ref[...], preferred_element_type=F32,
        precision=lax.Precision.HIGHEST)


def _tc_matscale(x, w, dinv):
    return pl.pallas_call(
        _matscale_body,
        grid=(NB,),
        in_specs=[
            pl.BlockSpec((128, D), lambda i: (i, 0)),
            pl.BlockSpec((D, D), lambda i: (0, 0)),
            pl.BlockSpec((128, 1), lambda i: (i, 0)),
        ],
        out_specs=pl.BlockSpec((128, D), lambda i: (i, 0)),
        out_shape=jax.ShapeDtypeStruct((NP, D), F32),
    )(x, w, dinv)


def _combine_body(aggA_ref, aggB_ref, dinv_ref, b_ref, wp_ref, h_ref, hwp_ref):
    dinv = dinv_ref[...]
    h = jnp.maximum(dinv * (aggA_ref[...] + aggB_ref[...]) + b_ref[...], 0.0)
    h_ref[...] = h
    hwp_ref[...] = dinv * jnp.dot(h, wp_ref[...], preferred_element_type=F32,
                                  precision=lax.Precision.HIGHEST)


def _tc_combine(aggA, aggB, dinv, brow, wp):
    return pl.pallas_call(
        _combine_body,
        grid=(NB,),
        in_specs=[
            pl.BlockSpec((128, D), lambda i: (i, 0)),
            pl.BlockSpec((128, D), lambda i: (i, 0)),
            pl.BlockSpec((128, 1), lambda i: (i, 0)),
            pl.BlockSpec((1, D), lambda i: (0, 0)),
            pl.BlockSpec((D, 1), lambda i: (0, 0)),
        ],
        out_specs=[
            pl.BlockSpec((128, D), lambda i: (i, 0)),
            pl.BlockSpec((128, 1), lambda i: (i, 0)),
        ],
        out_shape=[
            jax.ShapeDtypeStruct((NP, D), F32),
            jax.ShapeDtypeStruct((NP, 1), F32),
        ],
    )(aggA, aggB, dinv, brow, wp)


def _rank_body(clo_ref, chi_ref, s_ref, tb_ref, btf_ref, nm_ref, ts_ref,
               h_ref, sc_ref, tbc_ref, btc_ref, nmc_ref, krow_ref,
               bc_ref, kept_ref, hp_ref):
    i = pl.program_id(0)
    lo = clo_ref[i]
    hi = chi_ref[i]
    sr = s_ref[...]
    tbr = tb_ref[...]
    btr = btf_ref[...]
    nmr = nm_ref[...]

    def cbody(c, acc):
        sc = sc_ref[pl.ds(c, 1), :]
        tbc = tbc_ref[pl.ds(c, 1), :]
        btc = btc_ref[pl.ds(c, 1), :]
        nmc = nmc_ref[pl.ds(c, 1), :]
        beat = (sc > sr) | ((sc == sr) & (tbc < tbr))
        m = (btc == btr) & (nmc > 0.0) & beat
        return acc + jnp.sum(jnp.where(m, 1.0, 0.0), axis=1, keepdims=True)

    bc = lax.fori_loop(lo, hi + 1, cbody, jnp.zeros((128, 1), F32))
    kr = jnp.zeros((128, 1), F32)
    for g in range(NG):
        kr = kr + jnp.where(btr == float(g), krow_ref[0, g], 0.0)
    kept = (nmr > 0.0) & (bc < kr)
    bc_ref[...] = bc
    kept_ref[...] = jnp.where(kept, 1.0, 0.0)
    hp_ref[...] = jnp.where(kept, h_ref[...] * ts_ref[...], 0.0)


def _tc_rank(clo, chi, s, tb, btf, nm, ts, h, krow):
    full1 = lambda a: pl.BlockSpec((NB, 128), lambda i: (0, 0))
    return pl.pallas_call(
        _rank_body,
        grid=(NB,),
        in_specs=[
            pl.BlockSpec(memory_space=pltpu.SMEM),
            pl.BlockSpec(memory_space=pltpu.SMEM),
            pl.BlockSpec((128, 1), lambda i: (i, 0)),
            pl.BlockSpec((128, 1), lambda i: (i, 0)),
            pl.BlockSpec((128, 1), lambda i: (i, 0)),
            pl.BlockSpec((128, 1), lambda i: (i, 0)),
            pl.BlockSpec((128, 1), lambda i: (i, 0)),
            pl.BlockSpec((128, D), lambda i: (i, 0)),
            pl.BlockSpec((NB, 128), lambda i: (0, 0)),
            pl.BlockSpec((NB, 128), lambda i: (0, 0)),
            pl.BlockSpec((NB, 128), lambda i: (0, 0)),
            pl.BlockSpec((NB, 128), lambda i: (0, 0)),
            pl.BlockSpec((1, 128), lambda i: (0, 0)),
        ],
        out_specs=[
            pl.BlockSpec((128, 1), lambda i: (i, 0)),
            pl.BlockSpec((128, 1), lambda i: (i, 0)),
            pl.BlockSpec((128, D), lambda i: (i, 0)),
        ],
        out_shape=[
            jax.ShapeDtypeStruct((NP, 1), F32),
            jax.ShapeDtypeStruct((NP, 1), F32),
            jax.ShapeDtypeStruct((NP, D), F32),
        ],
    )(clo, chi, s, tb, btf, nm, ts, h,
      s.reshape(NB, 128), tb.reshape(NB, 128), btf.reshape(NB, 128),
      nm.reshape(NB, 128), krow)


def _readout_body(hp_ref, kept_ref, btf_ref, mx_ref, sm_ref, cnt_ref):
    i = pl.program_id(0)

    @pl.when(i == 0)
    def _():
        mx_ref[...] = jnp.full_like(mx_ref, -jnp.inf)
        sm_ref[...] = jnp.zeros_like(sm_ref)
        cnt_ref[...] = jnp.zeros_like(cnt_ref)

    btr = btf_ref[...]
    kr = kept_ref[...]
    hp = hp_ref[...]
    g0 = btr[0, 0].astype(I32)
    g1 = btr[127, 0].astype(I32)

    def body(g, _):
        gm = (btr == g.astype(F32)) & (kr > 0.0)
        red = jnp.max(jnp.where(gm, hp, -jnp.inf), axis=0, keepdims=True)
        mx_ref[pl.ds(g, 1), :] = jnp.maximum(mx_ref[pl.ds(g, 1), :], red)
        sums = jnp.sum(jnp.where(gm, hp, 0.0), axis=0, keepdims=True)
        sm_ref[pl.ds(g, 1), :] = sm_ref[pl.ds(g, 1), :] + sums
        cval = jnp.sum(jnp.where(gm, 1.0, 0.0))
        cnt_ref[pl.ds(g, 1), :] = cnt_ref[pl.ds(g, 1), :] + cval
        return 0

    lax.fori_loop(g0, g1 + 1, body, 0)


def _tc_readout(hp, kept, btf):
    return pl.pallas_call(
        _readout_body,
        grid=(NB,),
        in_specs=[
            pl.BlockSpec((128, D), lambda i: (i, 0)),
            pl.BlockSpec((128, 1), lambda i: (i, 0)),
            pl.BlockSpec((128, 1), lambda i: (i, 0)),
        ],
        out_specs=[
            pl.BlockSpec((NG, 128), lambda i: (0, 0)),
            pl.BlockSpec((NG, 128), lambda i: (0, 0)),
            pl.BlockSpec((NG, 128), lambda i: (0, 0)),
        ],
        out_shape=[
            jax.ShapeDtypeStruct((NG, 128), F32),
            jax.ShapeDtypeStruct((NG, 128), F32),
            jax.ShapeDtypeStruct((NG, 128), F32),
        ],
    )(hp, kept, btf)


def _mlp_body(mx1, sm1, c1, mx2, sm2, c2, mx3, sm3, c3,
              wl1, bl1, wl2, bl2, wl3, bl3, o_ref):
    left = mx1[...] + mx2[...] + mx3[...]
    right = sm1[...] / c1[...] + sm2[...] / c2[...] + sm3[...] / c3[...]
    hi = lax.Precision.HIGHEST
    z = left @ wl1[0:D, :] + right @ wl1[D:2 * D, :] + bl1[...]
    z = jnp.maximum(z, 0.0)
    z = jnp.maximum(jnp.dot(z, wl2[...], precision=hi) + bl2[...], 0.0)
    z = jnp.dot(z, wl3[...], precision=hi) + bl3[...]
    m = jnp.max(z, axis=1, keepdims=True)
    lse = jnp.log(jnp.sum(jnp.exp(z - m), axis=1, keepdims=True))
    o_ref[...] = z - m - lse


def _tc_mlp(parts, wl1, bl1, wl2, bl2, wl3, bl3):
    args = []
    for mx, sm, cnt in parts:
        args += [mx, sm, cnt]
    args += [wl1, bl1, wl2, bl2, wl3, bl3]
    return pl.pallas_call(
        _mlp_body,
        out_shape=jax.ShapeDtypeStruct((NG, 10), F32),
    )(*args)


@jax.jit
def kernel(x, edge_index, batch, W1, b1, Wp1, bp1, W2, b2, Wp2, bp2,
           W3, b3, Wp3, bp3, Wl1, bl1, Wl2, bl2, Wl3, bl3):
    pad_e = EP - E - N
    sl = jnp.arange(N, dtype=I32)
    srcE = jnp.concatenate([edge_index[0].astype(I32), sl,
                            jnp.zeros((pad_e,), I32)])
    dstE = jnp.concatenate([edge_index[1].astype(I32), sl,
                            jnp.full((pad_e,), TRASH, I32)])
    pmE = jnp.concatenate([jnp.ones((E + N,), F32), jnp.zeros((pad_e,), F32)])
    src3 = srcE.reshape(NW, EW)
    dst3 = dstE.reshape(NW, EW)
    pm3 = pmE.reshape(NW, EW)
    src3c = srcE.reshape(NW, CH, CW)
    dst3c = dstE.reshape(NW, CH, CW)
    d16 = dstE.reshape(NW, EW // 16, 16)

    z16 = jnp.zeros((ACC_H, 16), F32)
    z128 = jnp.zeros((ACC_H, D), F32)

    idx = jnp.arange(NP, dtype=I32)
    kept = jnp.where(idx < N, 1.0, 0.0).astype(F32).reshape(NP, 1)
    tb = idx.astype(F32).reshape(NP, 1)
    btp = jnp.concatenate([batch.astype(I32), jnp.full((NP - N,), NG - 1, I32)])
    btf = btp.astype(F32).reshape(NP, 1)
    X = jnp.zeros((NP, D), F32).at[:N].set(x)

    cnt_all_f = _tc_counts(btf, kept)
    c_all = cnt_all_f[:, 0]
    c_all_i = c_all.astype(I32)
    start = jnp.cumsum(c_all_i) - c_all_i
    g0 = btp[0::128]
    g1 = btp[127::128]
    clo = (start[g0] // 128).astype(I32)
    chi = ((start[g1] + c_all_i[g1] - 1) // 128).astype(I32)

    parts = []
    c_cur = c_all
    for (W, b, Wp, bp) in ((W1, b1, Wp1, bp1), (W2, b2, Wp2, bp2),
                           (W3, b3, Wp3, bp3)):
        w_e, degA, degB = _sc_k1(src3, pm3, d16, kept[:, 0], z16)
        deg = (degA + degB)[:NP, 0:1]
        dinv = jnp.where(deg > 0, deg ** -0.5, 0.0)
        hws = _tc_matscale(X, W, dinv)
        aggA, aggB = _sc_k2(hws, src3c, dst3c, z128)
        h, hwp = _tc_combine(aggA, aggB, dinv, b.reshape(1, D),
                             Wp.reshape(D, 1))
        sA, sB = _sc_k3(hwp[:, 0], src3, w_e, d16, z16)
        s = dinv * (sA + sB)[:NP, 0:1] + bp[0]
        ts = jnp.tanh(s)
        k_g = (4 * c_cur.astype(I32) + 4) // 5
        krow = jnp.zeros((1, 128), F32).at[0, :NG].set(k_g.astype(F32))
        tb, kept, hp = _tc_rank(clo, chi, s, tb, btf, kept, ts, h, krow)
        mx, sm, cnt = _tc_readout(hp, kept, btf)
        parts.append((mx[:, :D], sm[:, :D], cnt))
        c_cur = cnt[:, 0]
        X = hp

    return _tc_mlp(parts, Wl1, bl1.reshape(1, D), Wl2, bl2.reshape(1, D // 2),
                   Wl3, bl3.reshape(1, 10))


# speedup vs baseline: 24.6083x; 1.1509x over previous
import functools

import jax
import jax.numpy as jnp
from jax import lax
from jax.experimental import pallas as pl
from jax.experimental.pallas import tpu as pltpu
from jax.experimental.pallas import tpu_sc as plsc

N = 10000
D = 128
NG = 64
NP = 10112
NB = NP // 128
E = 320000
NW = 32
CH = 81
CW = 128
EP = NW * CH * CW
EW = CH * CW
CW2 = 64
CH2 = EW // CW2
ACC_H = 10240
TRASH = NP
F32 = jnp.float32
I32 = jnp.int32

_MESH = dict(core_axis_name="c", subcore_axis_name="s", num_cores=2,
             num_subcores=16)


def _k1_body(src_h, pm_h, dc_h, kept_h, z16_h, wo_h, degA_h, degB_h,
             kept_v, src_v, pm_v, wo_v, dc_v, rows128, acc):
    cid = lax.axis_index("c")
    sid = lax.axis_index("s")
    wid = cid * 16 + sid
    pltpu.sync_copy(z16_h.at[pl.ds(sid * 640, 640)], acc.at[pl.ds(sid * 640, 640)])
    pltpu.sync_copy(z16_h.at[pl.ds(0, 128)], rows128)
    pltpu.sync_copy(kept_h, kept_v)
    pltpu.sync_copy(src_h.at[wid], src_v)
    pltpu.sync_copy(pm_h.at[wid], pm_v)
    pltpu.sync_copy(dc_h.at[wid], dc_v)
    plsc.subcore_barrier()
    lanes = lax.iota(I32, 16)
    zcol = jnp.zeros((16,), I32)

    def chunk(c, _):
        base = c * CW
        for j in range(8):
            o = base + j * 16
            sv = src_v[pl.ds(o, 16)]
            dv = dc_v[c, pl.ds(j * 16, 16)]
            pm = pm_v[pl.ds(o, 16)]
            ks = plsc.load_gather(kept_v, [sv])
            kd = plsc.load_gather(kept_v, [dv])
            wn = pm * ks * kd
            wo_v[pl.ds(o, 16)] = wn
            plsc.store_scatter(rows128, [lanes + j * 16, zcol], wn)
        pltpu.sync_copy(rows128, acc.at[dc_v.at[c]], add=True)
        return 0

    lax.fori_loop(0, CH, chunk, 0)
    pltpu.sync_copy(wo_v, wo_h.at[wid])
    plsc.subcore_barrier()

    @pl.when(jnp.logical_and(sid == 0, cid == 0))
    def _():
        pltpu.sync_copy(acc, degA_h)

    @pl.when(jnp.logical_and(sid == 0, cid == 1))
    def _():
        pltpu.sync_copy(acc, degB_h)


def _sc_k1(src3, pm3, dst3c, kept, z16):
    fn = pl.kernel(
        _k1_body,
        out_type=(
            jax.ShapeDtypeStruct((NW, EW), F32),
            jax.ShapeDtypeStruct((ACC_H, 16), F32),
            jax.ShapeDtypeStruct((ACC_H, 16), F32),
        ),
        mesh=plsc.VectorSubcoreMesh(**_MESH),
        compiler_params=pltpu.CompilerParams(needs_layout_passes=False, use_tc_tiling_on_sc=False),
        scratch_types=(
            pltpu.VMEM((NP,), F32),
            pltpu.VMEM((EW,), I32),
            pltpu.VMEM((EW,), F32),
            pltpu.VMEM((EW,), F32),
            pltpu.VMEM((CH, CW), I32),
            pltpu.VMEM((CW, 16), F32),
            pltpu.VMEM_SHARED((ACC_H, 16), F32),
        ),
    )
    return fn(src3, pm3, dst3c, kept, z16)


def _k2_body(hws_h, src_h, dst_h, z128_h, aggA_h, aggB_h,
             src_v, dst_v, rows_a, rows_b, acc, sem_a, sem_b):
    cid = lax.axis_index("c")
    sid = lax.axis_index("s")
    wid = cid * 16 + sid
    pltpu.sync_copy(z128_h.at[pl.ds(sid * 640, 640)], acc.at[pl.ds(sid * 640, 640)])
    pltpu.sync_copy(src_h.at[wid], src_v)
    pltpu.sync_copy(dst_h.at[wid], dst_v)
    plsc.subcore_barrier()

    pltpu.async_copy(hws_h.at[src_v.at[0]], rows_a, sem_a)

    def pair(p, _):
        c0 = 2 * p
        c1 = c0 + 1
        c2 = c0 + 2

        @pl.when(c1 < CH2)
        def _():
            pltpu.async_copy(hws_h.at[src_v.at[c1]], rows_b, sem_b)

        pltpu.make_async_copy(hws_h.at[src_v.at[c0]], rows_a, sem_a).wait()
        pltpu.sync_copy(rows_a, acc.at[dst_v.at[c0]], add=True)

        @pl.when(c2 < CH2)
        def _():
            pltpu.async_copy(hws_h.at[src_v.at[c2]], rows_a, sem_a)

        @pl.when(c1 < CH2)
        def _():
            pltpu.make_async_copy(hws_h.at[src_v.at[c1]], rows_b, sem_b).wait()
            pltpu.sync_copy(rows_b, acc.at[dst_v.at[c1]], add=True)

        return 0

    lax.fori_loop(0, (CH2 + 1) // 2, pair, 0)
    plsc.subcore_barrier()

    @pl.when(jnp.logical_and(sid == 0, cid == 0))
    def _():
        pltpu.sync_copy(acc, aggA_h)

    @pl.when(jnp.logical_and(sid == 0, cid == 1))
    def _():
        pltpu.sync_copy(acc, aggB_h)


def _sc_k2(hws, src3c, dst3c, z128):
    fn = pl.kernel(
        _k2_body,
        out_type=(
            jax.ShapeDtypeStruct((ACC_H, D), F32),
            jax.ShapeDtypeStruct((ACC_H, D), F32),
        ),
        mesh=plsc.VectorSubcoreMesh(**_MESH),
        compiler_params=pltpu.CompilerParams(needs_layout_passes=False, use_tc_tiling_on_sc=False),
        scratch_types=(
            pltpu.VMEM((CH2, CW2), I32),
            pltpu.VMEM((CH2, CW2), I32),
            pltpu.VMEM((CW2, D), F32),
            pltpu.VMEM((CW2, D), F32),
            pltpu.VMEM_SHARED((ACC_H, D), F32),
            pltpu.SemaphoreType.DMA,
            pltpu.SemaphoreType.DMA,
        ),
    )
    return fn(hws, src3c, dst3c, z128)


def _k3_body(tab_h, src_h, w_h, dc_h, z16_h, sA_h, sB_h,
             tab_v, src_v, w_v, dc_v, rows128, acc):
    cid = lax.axis_index("c")
    sid = lax.axis_index("s")
    wid = cid * 16 + sid
    pltpu.sync_copy(z16_h.at[pl.ds(sid * 640, 640)], acc.at[pl.ds(sid * 640, 640)])
    pltpu.sync_copy(z16_h.at[pl.ds(0, 128)], rows128)
    pltpu.sync_copy(tab_h, tab_v)
    pltpu.sync_copy(src_h.at[wid], src_v)
    pltpu.sync_copy(w_h.at[wid], w_v)
    pltpu.sync_copy(dc_h.at[wid], dc_v)
    plsc.subcore_barrier()
    lanes = lax.iota(I32, 16)
    zcol = jnp.zeros((16,), I32)

    def chunk(c, _):
        base = c * CW
        for j in range(8):
            o = base + j * 16
            sv = src_v[pl.ds(o, 16)]
            wv = w_v[pl.ds(o, 16)]
            val = wv * plsc.load_gather(tab_v, [sv])
            plsc.store_scatter(rows128, [lanes + j * 16, zcol], val)
        pltpu.sync_copy(rows128, acc.at[dc_v.at[c]], add=True)
        return 0

    lax.fori_loop(0, CH, chunk, 0)
    plsc.subcore_barrier()

    @pl.when(jnp.logical_and(sid == 0, cid == 0))
    def _():
        pltpu.sync_copy(acc, sA_h)

    @pl.when(jnp.logical_and(sid == 0, cid == 1))
    def _():
        pltpu.sync_copy(acc, sB_h)


def _sc_k3(tab, src3, w3, dst3c, z16):
    fn = pl.kernel(
        _k3_body,
        out_type=(
            jax.ShapeDtypeStruct((ACC_H, 16), F32),
            jax.ShapeDtypeStruct((ACC_H, 16), F32),
        ),
        mesh=plsc.VectorSubcoreMesh(**_MESH),
        compiler_params=pltpu.CompilerParams(needs_layout_passes=False, use_tc_tiling_on_sc=False),
        scratch_types=(
            pltpu.VMEM((NP,), F32),
            pltpu.VMEM((EW,), I32),
            pltpu.VMEM((EW,), F32),
            pltpu.VMEM((CH, CW), I32),
            pltpu.VMEM((CW, 16), F32),
            pltpu.VMEM_SHARED((ACC_H, 16), F32),
        ),
    )
    return fn(tab, src3, w3, dst3c, z16)


def _counts_body(btf_ref, kept_ref, cnt_ref):
    i = pl.program_id(0)

    @pl.when(i == 0)
    def _():
        cnt_ref[...] = jnp.zeros_like(cnt_ref)

    btr = btf_ref[...]
    kr = kept_ref[...]
    for g in range(NG):
        val = jnp.sum(jnp.where(btr == float(g), kr, 0.0))
        cnt_ref[g:g + 1, :] = cnt_ref[g:g + 1, :] + val


def _tc_counts(btf, kept):
    return pl.pallas_call(
        _counts_body,
        grid=(NB,),
        in_specs=[
            pl.BlockSpec((128, 1), lambda i: (i, 0)),
            pl.BlockSpec((128, 1), lambda i: (i, 0)),
        ],
        out_specs=pl.BlockSpec((NG, 128), lambda i: (0, 0)),
        out_shape=jax.ShapeDtypeStruct((NG, 128), F32),
    )(btf, kept)


def _matscale_body(x_ref, w_ref, dinv_ref, o_ref):
    o_ref[...] = dinv_ref[...] * jnp.dot(
        x_ref[...], w_ref[...], preferred_element_type=F32,
        precision=lax.Precision.HIGHEST)


def _tc_matscale(x, w, dinv):
    return pl.pallas_call(
        _matscale_body,
        grid=(NB,),
        in_specs=[
            pl.BlockSpec((128, D), lambda i: (i, 0)),
            pl.BlockSpec((D, D), lambda i: (0, 0)),
            pl.BlockSpec((128, 1), lambda i: (i, 0)),
        ],
        out_specs=pl.BlockSpec((128, D), lambda i: (i, 0)),
        out_shape=jax.ShapeDtypeStruct((NP, D), F32),
    )(x, w, dinv)


def _combine_body(aggA_ref, aggB_ref, dinv_ref, b_ref, wp_ref, h_ref, hwp_ref):
    dinv = dinv_ref[...]
    h = jnp.maximum(dinv * (aggA_ref[...] + aggB_ref[...]) + b_ref[...], 0.0)
    h_ref[...] = h
    hwp_ref[...] = dinv * jnp.dot(h, wp_ref[...], preferred_element_type=F32,
                                  precision=lax.Precision.HIGHEST)


def _tc_combine(aggA, aggB, dinv, brow, wp):
    return pl.pallas_call(
        _combine_body,
        grid=(NB,),
        in_specs=[
            pl.BlockSpec((128, D), lambda i: (i, 0)),
            pl.BlockSpec((128, D), lambda i: (i, 0)),
            pl.BlockSpec((128, 1), lambda i: (i, 0)),
            pl.BlockSpec((1, D), lambda i: (0, 0)),
            pl.BlockSpec((D, 1), lambda i: (0, 0)),
        ],
        out_specs=[
            pl.BlockSpec((128, D), lambda i: (i, 0)),
            pl.BlockSpec((128, 1), lambda i: (i, 0)),
        ],
        out_shape=[
            jax.ShapeDtypeStruct((NP, D), F32),
            jax.ShapeDtypeStruct((NP, 1), F32),
        ],
    )(aggA, aggB, dinv, brow, wp)


def _rank_body(clo_ref, chi_ref, s_ref, tb_ref, btf_ref, nm_ref, ts_ref,
               h_ref, sc_ref, tbc_ref, btc_ref, nmc_ref, krow_ref,
               bc_ref, kept_ref, hp_ref):
    i = pl.program_id(0)
    lo = clo_ref[i]
    hi = chi_ref[i]
    sr = s_ref[...]
    tbr = tb_ref[...]
    btr = btf_ref[...]
    nmr = nm_ref[...]

    def cbody(c, acc):
        sc = sc_ref[pl.ds(c, 1), :]
        tbc = tbc_ref[pl.ds(c, 1), :]
        btc = btc_ref[pl.ds(c, 1), :]
        nmc = nmc_ref[pl.ds(c, 1), :]
        beat = (sc > sr) | ((sc == sr) & (tbc < tbr))
        m = (btc == btr) & (nmc > 0.0) & beat
        return acc + jnp.sum(jnp.where(m, 1.0, 0.0), axis=1, keepdims=True)

    bc = lax.fori_loop(lo, hi + 1, cbody, jnp.zeros((128, 1), F32))
    kr = jnp.zeros((128, 1), F32)
    for g in range(NG):
        kr = kr + jnp.where(btr == float(g), krow_ref[0, g], 0.0)
    kept = (nmr > 0.0) & (bc < kr)
    bc_ref[...] = bc
    kept_ref[...] = jnp.where(kept, 1.0, 0.0)
    hp_ref[...] = jnp.where(kept, h_ref[...] * ts_ref[...], 0.0)


def _tc_rank(clo, chi, s, tb, btf, nm, ts, h, krow):
    full1 = lambda a: pl.BlockSpec((NB, 128), lambda i: (0, 0))
    return pl.pallas_call(
        _rank_body,
        grid=(NB,),
        in_specs=[
            pl.BlockSpec(memory_space=pltpu.SMEM),
            pl.BlockSpec(memory_space=pltpu.SMEM),
            pl.BlockSpec((128, 1), lambda i: (i, 0)),
            pl.BlockSpec((128, 1), lambda i: (i, 0)),
            pl.BlockSpec((128, 1), lambda i: (i, 0)),
            pl.BlockSpec((128, 1), lambda i: (i, 0)),
            pl.BlockSpec((128, 1), lambda i: (i, 0)),
            pl.BlockSpec((128, D), lambda i: (i, 0)),
            pl.BlockSpec((NB, 128), lambda i: (0, 0)),
            pl.BlockSpec((NB, 128), lambda i: (0, 0)),
            pl.BlockSpec((NB, 128), lambda i: (0, 0)),
            pl.BlockSpec((NB, 128), lambda i: (0, 0)),
            pl.BlockSpec((1, 128), lambda i: (0, 0)),
        ],
        out_specs=[
            pl.BlockSpec((128, 1), lambda i: (i, 0)),
            pl.BlockSpec((128, 1), lambda i: (i, 0)),
            pl.BlockSpec((128, D), lambda i: (i, 0)),
        ],
        out_shape=[
            jax.ShapeDtypeStruct((NP, 1), F32),
            jax.ShapeDtypeStruct((NP, 1), F32),
            jax.ShapeDtypeStruct((NP, D), F32),
        ],
    )(clo, chi, s, tb, btf, nm, ts, h,
      s.reshape(NB, 128), tb.reshape(NB, 128), btf.reshape(NB, 128),
      nm.reshape(NB, 128), krow)


def _readout_body(hp_ref, kept_ref, btf_ref, mx_ref, sm_ref, cnt_ref):
    i = pl.program_id(0)

    @pl.when(i == 0)
    def _():
        mx_ref[...] = jnp.full_like(mx_ref, -jnp.inf)
        sm_ref[...] = jnp.zeros_like(sm_ref)
        cnt_ref[...] = jnp.zeros_like(cnt_ref)

    btr = btf_ref[...]
    kr = kept_ref[...]
    hp = hp_ref[...]
    g0 = btr[0, 0].astype(I32)
    g1 = btr[127, 0].astype(I32)

    def body(g, _):
        gm = (btr == g.astype(F32)) & (kr > 0.0)
        red = jnp.max(jnp.where(gm, hp, -jnp.inf), axis=0, keepdims=True)
        mx_ref[pl.ds(g, 1), :] = jnp.maximum(mx_ref[pl.ds(g, 1), :], red)
        sums = jnp.sum(jnp.where(gm, hp, 0.0), axis=0, keepdims=True)
        sm_ref[pl.ds(g, 1), :] = sm_ref[pl.ds(g, 1), :] + sums
        cval = jnp.sum(jnp.where(gm, 1.0, 0.0))
        cnt_ref[pl.ds(g, 1), :] = cnt_ref[pl.ds(g, 1), :] + cval
        return 0

    lax.fori_loop(g0, g1 + 1, body, 0)


def _tc_readout(hp, kept, btf):
    return pl.pallas_call(
        _readout_body,
        grid=(NB,),
        in_specs=[
            pl.BlockSpec((128, D), lambda i: (i, 0)),
            pl.BlockSpec((128, 1), lambda i: (i, 0)),
            pl.BlockSpec((128, 1), lambda i: (i, 0)),
        ],
        out_specs=[
            pl.BlockSpec((NG, 128), lambda i: (0, 0)),
            pl.BlockSpec((NG, 128), lambda i: (0, 0)),
            pl.BlockSpec((NG, 128), lambda i: (0, 0)),
        ],
        out_shape=[
            jax.ShapeDtypeStruct((NG, 128), F32),
            jax.ShapeDtypeStruct((NG, 128), F32),
            jax.ShapeDtypeStruct((NG, 128), F32),
        ],
    )(hp, kept, btf)


def _mlp_body(mx1, sm1, c1, mx2, sm2, c2, mx3, sm3, c3,
              wl1, bl1, wl2, bl2, wl3, bl3, o_ref):
    left = mx1[...] + mx2[...] + mx3[...]
    right = sm1[...] / c1[...] + sm2[...] / c2[...] + sm3[...] / c3[...]
    hi = lax.Precision.HIGHEST
    z = left @ wl1[0:D, :] + right @ wl1[D:2 * D, :] + bl1[...]
    z = jnp.maximum(z, 0.0)
    z = jnp.maximum(jnp.dot(z, wl2[...], precision=hi) + bl2[...], 0.0)
    z = jnp.dot(z, wl3[...], precision=hi) + bl3[...]
    m = jnp.max(z, axis=1, keepdims=True)
    lse = jnp.log(jnp.sum(jnp.exp(z - m), axis=1, keepdims=True))
    o_ref[...] = z - m - lse


def _tc_mlp(parts, wl1, bl1, wl2, bl2, wl3, bl3):
    args = []
    for mx, sm, cnt in parts:
        args += [mx, sm, cnt]
    args += [wl1, bl1, wl2, bl2, wl3, bl3]
    return pl.pallas_call(
        _mlp_body,
        out_shape=jax.ShapeDtypeStruct((NG, 10), F32),
    )(*args)


@jax.jit
def kernel(x, edge_index, batch, W1, b1, Wp1, bp1, W2, b2, Wp2, bp2,
           W3, b3, Wp3, bp3, Wl1, bl1, Wl2, bl2, Wl3, bl3):
    pad_e = EP - E - N
    sl = jnp.arange(N, dtype=I32)
    srcE = jnp.concatenate([edge_index[0].astype(I32), sl,
                            jnp.zeros((pad_e,), I32)])
    dstE = jnp.concatenate([edge_index[1].astype(I32), sl,
                            jnp.full((pad_e,), TRASH, I32)])
    pmE = jnp.concatenate([jnp.ones((E + N,), F32), jnp.zeros((pad_e,), F32)])
    src3 = srcE.reshape(NW, EW)
    dst3 = dstE.reshape(NW, EW)
    pm3 = pmE.reshape(NW, EW)
    src3c = srcE.reshape(NW, CH2, CW2)
    dst3c = dstE.reshape(NW, CH2, CW2)
    dst3k = dstE.reshape(NW, CH, CW)

    z16 = jnp.zeros((ACC_H, 16), F32)
    z128 = jnp.zeros((ACC_H, D), F32)

    idx = jnp.arange(NP, dtype=I32)
    kept = jnp.where(idx < N, 1.0, 0.0).astype(F32).reshape(NP, 1)
    tb = idx.astype(F32).reshape(NP, 1)
    btp = jnp.concatenate([batch.astype(I32), jnp.full((NP - N,), NG - 1, I32)])
    btf = btp.astype(F32).reshape(NP, 1)
    X = jnp.zeros((NP, D), F32).at[:N].set(x)

    cnt_all_f = _tc_counts(btf, kept)
    c_all = cnt_all_f[:, 0]
    c_all_i = c_all.astype(I32)
    start = jnp.cumsum(c_all_i) - c_all_i
    g0 = btp[0::128]
    g1 = btp[127::128]
    clo = (start[g0] // 128).astype(I32)
    chi = ((start[g1] + c_all_i[g1] - 1) // 128).astype(I32)

    parts = []
    c_cur = c_all
    for (W, b, Wp, bp) in ((W1, b1, Wp1, bp1), (W2, b2, Wp2, bp2),
                           (W3, b3, Wp3, bp3)):
        w_e, degA, degB = _sc_k1(src3, pm3, dst3k, kept[:, 0], z16)
        deg = (degA + degB)[:NP, 0:1]
        dinv = jnp.where(deg > 0, deg ** -0.5, 0.0)
        hws = _tc_matscale(X, W, dinv)
        aggA, aggB = _sc_k2(hws, src3c, dst3c, z128)
        h, hwp = _tc_combine(aggA, aggB, dinv, b.reshape(1, D),
                             Wp.reshape(D, 1))
        sA, sB = _sc_k3(hwp[:, 0], src3, w_e, dst3k, z16)
        s = dinv * (sA + sB)[:NP, 0:1] + bp[0]
        ts = jnp.tanh(s)
        k_g = (4 * c_cur.astype(I32) + 4) // 5
        krow = jnp.zeros((1, 128), F32).at[0, :NG].set(k_g.astype(F32))
        tb, kept, hp = _tc_rank(clo, chi, s, tb, btf, kept, ts, h, krow)
        mx, sm, cnt = _tc_readout(hp, kept, btf)
        parts.append((mx[:, :D], sm[:, :D], cnt))
        c_cur = cnt[:, 0]
        X = hp

    return _tc_mlp(parts, Wl1, bl1.reshape(1, D), Wl2, bl2.reshape(1, D // 2),
                   Wl3, bl3.reshape(1, 10))
